# baseline (speedup 1.0000x reference)
import functools

import jax
import jax.numpy as jnp
from jax import lax
from jax.experimental import pallas as pl
from jax.experimental.pallas import tpu as pltpu
from jax.experimental.pallas import tpu_sc as plsc

N = 10000
E = 320000
H = 128
NC = 5
NL = 2
G = 64

NUM_SC = 2
NUM_TILES = 16
NW = NUM_SC * NUM_TILES

CH = 32
CPT = 320
EPAD = NW * CPT * CH

NP = 10240
RPT = NP // NUM_TILES

HALF = CPT // 8


def _sc_agg_body(h_hbm, src_hbm, dst_hbm, out_hbm, sslab, dslab, gbufs, sems,
                 agg_sh):
    cid = lax.axis_index("c")
    sid = lax.axis_index("s")
    wid = sid * NUM_SC + cid

    zbuf = gbufs.at[0]

    @pl.loop(0, CH)
    def _(r):
        @pl.loop(0, H, step=16)
        def _(j):
            zbuf.at[r, pl.ds(j, 16)][...] = jnp.zeros((16,), jnp.float32)

    @pl.loop(0, RPT // CH)
    def _(k):
        pltpu.sync_copy(zbuf, agg_sh.at[pl.ds(sid * RPT + k * CH, CH)])

    plsc.subcore_barrier()

    @pl.loop(0, 8)
    def _(half):
        base = wid * CPT + half * HALF
        pltpu.sync_copy(src_hbm.at[pl.ds(base, HALF)], sslab)
        pltpu.sync_copy(dst_hbm.at[pl.ds(base, HALF)], dslab)
        for p in range(8):
            pltpu.async_copy(h_hbm.at[sslab.at[p]], gbufs.at[p], sems.at[p])

        @pl.loop(0, HALF - 8, step=8)
        def _(j):
            for p in range(8):
                jj = j + p
                pltpu.make_async_copy(h_hbm.at[sslab.at[jj]], gbufs.at[p],
                                      sems.at[p]).wait()
                pltpu.sync_copy(gbufs.at[p], agg_sh.at[dslab.at[jj]],
                                add=True)
                pltpu.async_copy(h_hbm.at[sslab.at[jj + 8]], gbufs.at[p],
                                 sems.at[p])

        for p in range(8):
            jj = HALF - 8 + p
            pltpu.make_async_copy(h_hbm.at[sslab.at[jj]], gbufs.at[p],
                                  sems.at[p]).wait()
            pltpu.sync_copy(gbufs.at[p], agg_sh.at[dslab.at[jj]], add=True)

    plsc.subcore_barrier()

    pltpu.sync_copy(agg_sh.at[pl.ds(sid * RPT, RPT)],
                    out_hbm.at[cid, pl.ds(sid * RPT, RPT)])


@functools.cache
def _get_sc_agg():
    mesh = plsc.VectorSubcoreMesh(
        core_axis_name="c", subcore_axis_name="s",
        num_cores=NUM_SC, num_subcores=NUM_TILES)
    return pl.kernel(
        _sc_agg_body,
        out_type=jax.ShapeDtypeStruct((NUM_SC, NP, H), jnp.float32),
        mesh=mesh,
        scratch_types=[
            pltpu.VMEM((HALF, CH), jnp.int32),
            pltpu.VMEM((HALF, CH), jnp.int32),
            pltpu.VMEM((8, CH, H), jnp.float32),
            pltpu.SemaphoreType.DMA((8,)),
            pltpu.VMEM_SHARED((NP, H), jnp.float32),
        ],
    )


BU = 1000


def _update_body(h_ref, a_ref, w_ref, b_ref, o_ref):
    hh = h_ref[...] + a_ref[0] + a_ref[1]
    o_ref[...] = jnp.maximum(
        jnp.dot(hh, w_ref[...], preferred_element_type=jnp.float32,
                precision=lax.Precision.HIGHEST)
        + b_ref[...], 0.0)


_update = pl.pallas_call(
    _update_body,
    grid=(N // BU,),
    in_specs=[
        pl.BlockSpec((BU, H), lambda i: (i, 0)),
        pl.BlockSpec((NUM_SC, BU, H), lambda i: (0, i, 0)),
        pl.BlockSpec((H, H), lambda i: (0, 0)),
        pl.BlockSpec((1, H), lambda i: (0, 0)),
    ],
    out_specs=pl.BlockSpec((BU, H), lambda i: (i, 0)),
    out_shape=jax.ShapeDtypeStruct((N, H), jnp.float32),
)


def _head_body(h_ref, batch_ref, wl_ref, bl_ref, wo_ref, bo_ref, o_ref):
    seg = lax.broadcasted_iota(jnp.int32, (G, N), 0)
    p = (batch_ref[...] == seg).astype(jnp.float32)
    sums = jnp.dot(p, h_ref[...], preferred_element_type=jnp.float32,
                   precision=lax.Precision.HIGHEST)
    counts = jnp.sum(p, axis=1, keepdims=True)
    g = jnp.maximum(sums / jnp.maximum(counts, 1.0), 0.0)
    for i in range(NL):
        g = jnp.maximum(
            jnp.dot(g, wl_ref[i], preferred_element_type=jnp.float32,
                    precision=lax.Precision.HIGHEST)
            + bl_ref[i], 0.0)
    o_ref[...] = (jnp.dot(g, wo_ref[...], preferred_element_type=jnp.float32,
                   precision=lax.Precision.HIGHEST)
                  + bo_ref[...])


_head = pl.pallas_call(
    _head_body,
    out_shape=jax.ShapeDtypeStruct((G, 1), jnp.float32),
)


def kernel(x, edge_index, batch, Wc, bc, Wl, bl, Wout, bout):
    src = edge_index[0]
    dst = edge_index[1]
    pad = EPAD - E
    srcp = jnp.concatenate([src, jnp.zeros((pad,), jnp.int32)]).reshape(-1, CH)
    dstp = jnp.concatenate([dst, jnp.full((pad,), N, jnp.int32)]).reshape(-1, CH)
    batch2 = batch.reshape(1, N)

    sc_agg = _get_sc_agg()
    h = x
    for i in range(NC):
        agg = sc_agg(h, srcp, dstp)
        h = _update(h, agg, Wc[i], bc[i].reshape(1, H))
    out = _head(h, batch2, Wl, bl, Wout, bout.reshape(1, 1))
    return out.reshape(-1)


# speedup vs baseline: 1.1614x; 1.1614x over previous
import functools

import jax
import jax.numpy as jnp
from jax import lax
from jax.experimental import pallas as pl
from jax.experimental.pallas import tpu as pltpu
from jax.experimental.pallas import tpu_sc as plsc

N = 10000
E = 320000
H = 128
NC = 5
NL = 2
G = 64

NUM_SC = 2
NUM_TILES = 16
NW = NUM_SC * NUM_TILES

CH = 64
CPT = 160
EPAD = NW * CPT * CH

NP = 10240
RPT = NP // NUM_TILES

HALF = CPT // 4


def _sc_agg_body(h_hbm, src_hbm, dst_hbm, out_hbm, sslab, dslab, gbufs, sems,
                 agg_sh):
    cid = lax.axis_index("c")
    sid = lax.axis_index("s")
    wid = sid * NUM_SC + cid

    zbuf = gbufs.at[0]

    @pl.loop(0, CH)
    def _(r):
        @pl.loop(0, H, step=16)
        def _(j):
            zbuf.at[r, pl.ds(j, 16)][...] = jnp.zeros((16,), jnp.float32)

    @pl.loop(0, RPT // CH)
    def _(k):
        pltpu.sync_copy(zbuf, agg_sh.at[pl.ds(sid * RPT + k * CH, CH)])

    plsc.subcore_barrier()

    @pl.loop(0, 4)
    def _(half):
        base = wid * CPT + half * HALF
        pltpu.sync_copy(src_hbm.at[pl.ds(base, HALF)], sslab)
        pltpu.sync_copy(dst_hbm.at[pl.ds(base, HALF)], dslab)
        for p in range(4):
            pltpu.async_copy(h_hbm.at[sslab.at[p]], gbufs.at[p], sems.at[p])

        @pl.loop(0, HALF - 4, step=4)
        def _(j):
            for p in range(4):
                jj = j + p
                pltpu.make_async_copy(h_hbm.at[sslab.at[jj]], gbufs.at[p],
                                      sems.at[p]).wait()
                pltpu.sync_copy(gbufs.at[p], agg_sh.at[dslab.at[jj]],
                                add=True)
                pltpu.async_copy(h_hbm.at[sslab.at[jj + 4]], gbufs.at[p],
                                 sems.at[p])

        for p in range(4):
            jj = HALF - 4 + p
            pltpu.make_async_copy(h_hbm.at[sslab.at[jj]], gbufs.at[p],
                                  sems.at[p]).wait()
            pltpu.sync_copy(gbufs.at[p], agg_sh.at[dslab.at[jj]], add=True)

    plsc.subcore_barrier()

    pltpu.sync_copy(agg_sh.at[pl.ds(sid * RPT, RPT)],
                    out_hbm.at[cid, pl.ds(sid * RPT, RPT)])


@functools.cache
def _get_sc_agg():
    mesh = plsc.VectorSubcoreMesh(
        core_axis_name="c", subcore_axis_name="s",
        num_cores=NUM_SC, num_subcores=NUM_TILES)
    return pl.kernel(
        _sc_agg_body,
        out_type=jax.ShapeDtypeStruct((NUM_SC, NP, H), jnp.float32),
        mesh=mesh,
        scratch_types=[
            pltpu.VMEM((HALF, CH), jnp.int32),
            pltpu.VMEM((HALF, CH), jnp.int32),
            pltpu.VMEM((4, CH, H), jnp.float32),
            pltpu.SemaphoreType.DMA((4,)),
            pltpu.VMEM_SHARED((NP, H), jnp.float32),
        ],
    )


BU = 1000


def _update_body(h_ref, a_ref, w_ref, b_ref, o_ref):
    hh = h_ref[...] + a_ref[0] + a_ref[1]
    o_ref[...] = jnp.maximum(
        jnp.dot(hh, w_ref[...], preferred_element_type=jnp.float32,
                precision=lax.Precision.HIGHEST)
        + b_ref[...], 0.0)


_update = pl.pallas_call(
    _update_body,
    grid=(N // BU,),
    in_specs=[
        pl.BlockSpec((BU, H), lambda i: (i, 0)),
        pl.BlockSpec((NUM_SC, BU, H), lambda i: (0, i, 0)),
        pl.BlockSpec((H, H), lambda i: (0, 0)),
        pl.BlockSpec((1, H), lambda i: (0, 0)),
    ],
    out_specs=pl.BlockSpec((BU, H), lambda i: (i, 0)),
    out_shape=jax.ShapeDtypeStruct((N, H), jnp.float32),
)


def _head_body(h_ref, batch_ref, wl_ref, bl_ref, wo_ref, bo_ref, o_ref):
    seg = lax.broadcasted_iota(jnp.int32, (G, N), 0)
    p = (batch_ref[...] == seg).astype(jnp.float32)
    sums = jnp.dot(p, h_ref[...], preferred_element_type=jnp.float32,
                   precision=lax.Precision.HIGHEST)
    counts = jnp.sum(p, axis=1, keepdims=True)
    g = jnp.maximum(sums / jnp.maximum(counts, 1.0), 0.0)
    for i in range(NL):
        g = jnp.maximum(
            jnp.dot(g, wl_ref[i], preferred_element_type=jnp.float32,
                    precision=lax.Precision.HIGHEST)
            + bl_ref[i], 0.0)
    o_ref[...] = (jnp.dot(g, wo_ref[...], preferred_element_type=jnp.float32,
                   precision=lax.Precision.HIGHEST)
                  + bo_ref[...])


_head = pl.pallas_call(
    _head_body,
    out_shape=jax.ShapeDtypeStruct((G, 1), jnp.float32),
)


def kernel(x, edge_index, batch, Wc, bc, Wl, bl, Wout, bout):
    src = edge_index[0]
    dst = edge_index[1]
    pad = EPAD - E
    srcp = jnp.concatenate([src, jnp.zeros((pad,), jnp.int32)]).reshape(-1, CH)
    dstp = jnp.concatenate([dst, jnp.full((pad,), N, jnp.int32)]).reshape(-1, CH)
    batch2 = batch.reshape(1, N)

    sc_agg = _get_sc_agg()
    h = x
    for i in range(NC):
        agg = sc_agg(h, srcp, dstp)
        h = _update(h, agg, Wc[i], bc[i].reshape(1, H))
    out = _head(h, batch2, Wl, bl, Wout, bout.reshape(1, 1))
    return out.reshape(-1)
